# baseline (speedup 1.0000x reference)
import functools

import jax
import jax.numpy as jnp
from jax import lax
from jax.experimental import pallas as pl
from jax.experimental.pallas import tpu as pltpu
from jax.experimental.pallas import tpu_sc as plsc

N = 10000
E = 320000
D = 128

NC = 2
NS = 16
NW = NC * NS

PB = 128
STEPS = 80
E_PAD = NW * STEPS * PB
S_FAST = 128
S_SLOW = 32
SCHUNK = 32
E_FAST = NS * S_FAST * PB
FAST_CID = 0
N_ACC = 10240
ZROWS = N_ACC // NS
DROWS = 624
DTAIL = N - NS * DROWS

_mesh = plsc.VectorSubcoreMesh(core_axis_name="c", subcore_axis_name="s")


@functools.partial(
    pl.kernel,
    mesh=_mesh,
    out_type=jax.ShapeDtypeStruct((NC, N, 16), jnp.float32),
    scratch_types=[
        pltpu.VMEM((STEPS, PB), jnp.int32),
        pltpu.VMEM((PB, 16), jnp.float32),
        pltpu.VMEM((PB, 16), jnp.float32),
        pltpu.VMEM_SHARED((N_ACC, 16), jnp.float32),
        pltpu.SemaphoreType.DMA,
    ],
)
def _sc_degree(dst_hbm, out_hbm, dst_v, ones_v, zero_v, acc, sem):
    cid = lax.axis_index("c")
    sid = lax.axis_index("s")
    wid = sid * NC + cid

    pltpu.async_copy(dst_hbm.at[wid], dst_v, sem)

    one = jnp.full((16,), 1.0, jnp.float32)
    zero = jnp.zeros((16,), jnp.float32)

    @pl.loop(0, PB)
    def _(i):
        ones_v[i, pl.ds(0, 16)] = one
        zero_v[i, pl.ds(0, 16)] = zero

    zbase = sid * ZROWS

    @pl.loop(0, ZROWS, step=PB)
    def _(r):
        pltpu.sync_copy(zero_v, acc.at[pl.ds(zbase + r, PB)])

    pltpu.make_async_copy(dst_hbm.at[wid], dst_v, sem).wait()
    plsc.subcore_barrier()

    @pl.loop(0, STEPS)
    def _(s):
        pltpu.sync_copy(ones_v, acc.at[dst_v.at[s]], add=True)

    plsc.subcore_barrier()
    dbase = sid * DROWS
    pltpu.sync_copy(acc.at[pl.ds(dbase, DROWS)],
                    out_hbm.at[cid].at[pl.ds(dbase, DROWS)])

    @pl.when(sid == 0)
    def _():
        pltpu.sync_copy(acc.at[pl.ds(NS * DROWS, DTAIL)],
                        out_hbm.at[cid].at[pl.ds(NS * DROWS, DTAIL)])


@functools.partial(
    pl.kernel,
    mesh=_mesh,
    out_type=jax.ShapeDtypeStruct((NC, N, D), jnp.float32),
    scratch_types=[
        pltpu.VMEM((SCHUNK, PB), jnp.int32),
        pltpu.VMEM((SCHUNK, PB), jnp.int32),
        pltpu.VMEM((PB, D), jnp.float32),
        pltpu.VMEM((PB, D), jnp.float32),
        pltpu.VMEM_SHARED((N_ACC, D), jnp.float32),
        pltpu.SemaphoreType.DMA,
        pltpu.SemaphoreType.DMA,
        pltpu.SemaphoreType.DMA,
    ],
)
def _sc_segsum(g_hbm, src_f, dst_f, src_s, dst_s, out_hbm,
               src_v, dst_v, buf0, buf1, acc, sem0, sem1, semi):
    cid = lax.axis_index("c")
    sid = lax.axis_index("s")

    zero = jnp.zeros((16,), jnp.float32)

    @pl.loop(0, PB)
    def _(i):
        @pl.loop(0, D, step=16)
        def _(j):
            buf0[i, pl.ds(j, 16)] = zero

    zbase = sid * ZROWS

    @pl.loop(0, ZROWS, step=PB)
    def _(r):
        pltpu.sync_copy(buf0, acc.at[pl.ds(zbase + r, PB)])

    plsc.subcore_barrier()

    def process_chunk(src_chunk_hbm, dst_chunk_hbm):
        pltpu.async_copy(src_chunk_hbm, src_v, semi)
        pltpu.async_copy(dst_chunk_hbm, dst_v, semi)
        pltpu.make_async_copy(src_chunk_hbm, src_v, semi).wait()
        pltpu.make_async_copy(dst_chunk_hbm, dst_v, semi).wait()

        pltpu.async_copy(g_hbm.at[src_v.at[0]], buf0, sem0)

        @pl.loop(0, SCHUNK - 2, step=2)
        def _(s):
            pltpu.async_copy(g_hbm.at[src_v.at[s + 1]], buf1, sem1)
            pltpu.make_async_copy(g_hbm.at[src_v.at[s]], buf0, sem0).wait()
            pltpu.sync_copy(buf0, acc.at[dst_v.at[s]], add=True)
            pltpu.async_copy(g_hbm.at[src_v.at[s + 2]], buf0, sem0)
            pltpu.make_async_copy(g_hbm.at[src_v.at[s + 1]], buf1, sem1).wait()
            pltpu.sync_copy(buf1, acc.at[dst_v.at[s + 1]], add=True)

        pltpu.async_copy(g_hbm.at[src_v.at[SCHUNK - 1]], buf1, sem1)
        pltpu.make_async_copy(g_hbm.at[src_v.at[SCHUNK - 2]], buf0, sem0).wait()
        pltpu.sync_copy(buf0, acc.at[dst_v.at[SCHUNK - 2]], add=True)
        pltpu.make_async_copy(g_hbm.at[src_v.at[SCHUNK - 1]], buf1, sem1).wait()
        pltpu.sync_copy(buf1, acc.at[dst_v.at[SCHUNK - 1]], add=True)

    @pl.when(cid == FAST_CID)
    def _():
        for h in range(S_FAST // SCHUNK):
            process_chunk(src_f.at[sid].at[pl.ds(h * SCHUNK, SCHUNK)],
                          dst_f.at[sid].at[pl.ds(h * SCHUNK, SCHUNK)])

    @pl.when(cid != FAST_CID)
    def _():
        for h in range(S_SLOW // SCHUNK):
            process_chunk(src_s.at[sid].at[pl.ds(h * SCHUNK, SCHUNK)],
                          dst_s.at[sid].at[pl.ds(h * SCHUNK, SCHUNK)])

    plsc.subcore_barrier()
    dbase = sid * DROWS
    pltpu.sync_copy(acc.at[pl.ds(dbase, DROWS)],
                    out_hbm.at[cid].at[pl.ds(dbase, DROWS)])

    @pl.when(sid == 0)
    def _():
        pltpu.sync_copy(acc.at[pl.ds(NS * DROWS, DTAIL)],
                        out_hbm.at[cid].at[pl.ds(NS * DROWS, DTAIL)])


def _tc_first(x, W0, b0):
    def body(x_ref, w_ref, b_ref, o_ref):
        o_ref[...] = jnp.maximum(
            jnp.dot(x_ref[...], w_ref[...], preferred_element_type=jnp.float32)
            + b_ref[...], 0.0)

    return pl.pallas_call(
        body, out_shape=jax.ShapeDtypeStruct((N, D), jnp.float32))(x, W0, b0)


def _dinv_from_parts(p_ref):
    deg = 1.0 + p_ref[0][:, 0:1] + p_ref[1][:, 0:1]
    return 1.0 / jnp.sqrt(deg)


def _tc_scale(h, W, degp):
    def body(h_ref, w_ref, p_ref, o_ref):
        dinv = _dinv_from_parts(p_ref)
        o_ref[...] = jnp.dot(
            h_ref[...], w_ref[...], preferred_element_type=jnp.float32) * dinv

    return pl.pallas_call(
        body, out_shape=jax.ShapeDtypeStruct((N, D), jnp.float32))(h, W, degp)


def _tc_mid(s, g, b, W, degp):
    def body(s_ref, g_ref, b_ref, w_ref, p_ref, o_ref):
        dinv = _dinv_from_parts(p_ref)
        h = jnp.maximum(dinv * (s_ref[0] + s_ref[1] + g_ref[...]) + b_ref[...],
                        0.0)
        o_ref[...] = jnp.dot(
            h, w_ref[...], preferred_element_type=jnp.float32) * dinv

    return pl.pallas_call(
        body, out_shape=jax.ShapeDtypeStruct((N, D), jnp.float32))(
            s, g, b, W, degp)


def _tc_last(s, g, b, degp):
    def body(s_ref, g_ref, b_ref, p_ref, o_ref):
        dinv = _dinv_from_parts(p_ref)
        o_ref[...] = dinv * (s_ref[0] + s_ref[1] + g_ref[...]) + b_ref[...]

    return pl.pallas_call(
        body, out_shape=jax.ShapeDtypeStruct((N, D), jnp.float32))(s, g, b, degp)


def kernel(x, edge_index, W0, b0, W1, b1, W2, b2):
    src = edge_index[0]
    dst = edge_index[1]
    pad = E_PAD - E
    src_p = jnp.concatenate([src, jnp.zeros((pad,), src.dtype)])
    dst_p = jnp.concatenate([dst, jnp.full((pad,), N, dst.dtype)])
    src_f = src_p[:E_FAST].reshape(NS, S_FAST, PB)
    dst_f = dst_p[:E_FAST].reshape(NS, S_FAST, PB)
    src_s = src_p[E_FAST:].reshape(NS, S_SLOW, PB)
    dst_s = dst_p[E_FAST:].reshape(NS, S_SLOW, PB)

    degp = _sc_degree(dst_p.reshape(NW, STEPS, PB))
    h0 = _tc_first(x, W0, b0.reshape(1, D))
    g1 = _tc_scale(h0, W1, degp)
    s1 = _sc_segsum(g1, src_f, dst_f, src_s, dst_s)
    g2 = _tc_mid(s1, g1, b1.reshape(1, D), W2, degp)
    s2 = _sc_segsum(g2, src_f, dst_f, src_s, dst_s)
    return _tc_last(s2, g2, b2.reshape(1, D), degp)


# speedup vs baseline: 1.0068x; 1.0068x over previous
import functools

import jax
import jax.numpy as jnp
from jax import lax
from jax.experimental import pallas as pl
from jax.experimental.pallas import tpu as pltpu
from jax.experimental.pallas import tpu_sc as plsc

N = 10000
E = 320000
D = 128

NC = 2
NS = 16
NW = NC * NS

PB = 128
STEPS = 80
E_PAD = NW * STEPS * PB
SCHUNK = 40
N_ACC = 10240
ZROWS = N_ACC // NS
DROWS = 624
DTAIL = N - NS * DROWS

_mesh = plsc.VectorSubcoreMesh(core_axis_name="c", subcore_axis_name="s")


@functools.partial(
    pl.kernel,
    mesh=_mesh,
    out_type=jax.ShapeDtypeStruct((NC, N, 16), jnp.float32),
    scratch_types=[
        pltpu.VMEM((STEPS, PB), jnp.int32),
        pltpu.VMEM((PB, 16), jnp.float32),
        pltpu.VMEM((PB, 16), jnp.float32),
        pltpu.VMEM_SHARED((N_ACC, 16), jnp.float32),
        pltpu.SemaphoreType.DMA,
    ],
)
def _sc_degree(dst_hbm, out_hbm, dst_v, ones_v, zero_v, acc, sem):
    cid = lax.axis_index("c")
    sid = lax.axis_index("s")
    wid = sid * NC + cid

    pltpu.async_copy(dst_hbm.at[wid], dst_v, sem)

    one = jnp.full((16,), 1.0, jnp.float32)
    zero = jnp.zeros((16,), jnp.float32)

    @pl.loop(0, PB)
    def _(i):
        ones_v[i, pl.ds(0, 16)] = one
        zero_v[i, pl.ds(0, 16)] = zero

    zbase = sid * ZROWS

    @pl.loop(0, ZROWS, step=PB)
    def _(r):
        pltpu.sync_copy(zero_v, acc.at[pl.ds(zbase + r, PB)])

    pltpu.make_async_copy(dst_hbm.at[wid], dst_v, sem).wait()
    plsc.subcore_barrier()

    @pl.loop(0, STEPS)
    def _(s):
        pltpu.sync_copy(ones_v, acc.at[dst_v.at[s]], add=True)

    plsc.subcore_barrier()
    dbase = sid * DROWS
    pltpu.sync_copy(acc.at[pl.ds(dbase, DROWS)],
                    out_hbm.at[cid].at[pl.ds(dbase, DROWS)])

    @pl.when(sid == 0)
    def _():
        pltpu.sync_copy(acc.at[pl.ds(NS * DROWS, DTAIL)],
                        out_hbm.at[cid].at[pl.ds(NS * DROWS, DTAIL)])


@functools.partial(
    pl.kernel,
    mesh=_mesh,
    out_type=jax.ShapeDtypeStruct((NC, N, D), jnp.float32),
    scratch_types=[
        pltpu.VMEM((SCHUNK, PB), jnp.int32),
        pltpu.VMEM((SCHUNK, PB), jnp.int32),
        pltpu.VMEM((PB, D), jnp.float32),
        pltpu.VMEM((PB, D), jnp.float32),
        pltpu.VMEM_SHARED((N_ACC, D), jnp.float32),
        pltpu.SemaphoreType.DMA,
        pltpu.SemaphoreType.DMA,
        pltpu.SemaphoreType.DMA,
    ],
)
def _sc_segsum(g_hbm, src_hbm, dst_hbm, out_hbm,
               src_v, dst_v, buf0, buf1, acc, sem0, sem1, semi):
    cid = lax.axis_index("c")
    sid = lax.axis_index("s")
    wid = sid * NC + cid

    zero = jnp.zeros((16,), jnp.float32)

    @pl.loop(0, PB)
    def _(i):
        @pl.loop(0, D, step=16)
        def _(j):
            buf0[i, pl.ds(j, 16)] = zero

    zbase = sid * ZROWS

    @pl.loop(0, ZROWS, step=PB)
    def _(r):
        pltpu.sync_copy(buf0, acc.at[pl.ds(zbase + r, PB)])

    plsc.subcore_barrier()

    def process_chunk(src_chunk_hbm, dst_chunk_hbm):
        pltpu.async_copy(src_chunk_hbm, src_v, semi)
        pltpu.async_copy(dst_chunk_hbm, dst_v, semi)
        pltpu.make_async_copy(src_chunk_hbm, src_v, semi).wait()
        pltpu.make_async_copy(dst_chunk_hbm, dst_v, semi).wait()

        pltpu.async_copy(g_hbm.at[src_v.at[0]], buf0, sem0)

        @pl.loop(0, SCHUNK - 2, step=2)
        def _(s):
            pltpu.async_copy(g_hbm.at[src_v.at[s + 1]], buf1, sem1)
            pltpu.make_async_copy(g_hbm.at[src_v.at[s]], buf0, sem0).wait()
            pltpu.sync_copy(buf0, acc.at[dst_v.at[s]], add=True)
            pltpu.async_copy(g_hbm.at[src_v.at[s + 2]], buf0, sem0)
            pltpu.make_async_copy(g_hbm.at[src_v.at[s + 1]], buf1, sem1).wait()
            pltpu.sync_copy(buf1, acc.at[dst_v.at[s + 1]], add=True)

        pltpu.async_copy(g_hbm.at[src_v.at[SCHUNK - 1]], buf1, sem1)
        pltpu.make_async_copy(g_hbm.at[src_v.at[SCHUNK - 2]], buf0, sem0).wait()
        pltpu.sync_copy(buf0, acc.at[dst_v.at[SCHUNK - 2]], add=True)
        pltpu.make_async_copy(g_hbm.at[src_v.at[SCHUNK - 1]], buf1, sem1).wait()
        pltpu.sync_copy(buf1, acc.at[dst_v.at[SCHUNK - 1]], add=True)

    for h in range(STEPS // SCHUNK):
        process_chunk(src_hbm.at[wid].at[pl.ds(h * SCHUNK, SCHUNK)],
                      dst_hbm.at[wid].at[pl.ds(h * SCHUNK, SCHUNK)])

    plsc.subcore_barrier()
    dbase = sid * DROWS
    pltpu.sync_copy(acc.at[pl.ds(dbase, DROWS)],
                    out_hbm.at[cid].at[pl.ds(dbase, DROWS)])

    @pl.when(sid == 0)
    def _():
        pltpu.sync_copy(acc.at[pl.ds(NS * DROWS, DTAIL)],
                        out_hbm.at[cid].at[pl.ds(NS * DROWS, DTAIL)])


def _tc_first(x, W0, b0):
    def body(x_ref, w_ref, b_ref, o_ref):
        o_ref[...] = jnp.maximum(
            jnp.dot(x_ref[...], w_ref[...], preferred_element_type=jnp.float32)
            + b_ref[...], 0.0)

    return pl.pallas_call(
        body, out_shape=jax.ShapeDtypeStruct((N, D), jnp.float32))(x, W0, b0)


def _dinv_from_parts(p_ref):
    deg = 1.0 + p_ref[0][:, 0:1] + p_ref[1][:, 0:1]
    return 1.0 / jnp.sqrt(deg)


def _tc_scale(h, W, degp):
    def body(h_ref, w_ref, p_ref, o_ref):
        dinv = _dinv_from_parts(p_ref)
        o_ref[...] = jnp.dot(
            h_ref[...], w_ref[...], preferred_element_type=jnp.float32) * dinv

    return pl.pallas_call(
        body, out_shape=jax.ShapeDtypeStruct((N, D), jnp.float32))(h, W, degp)


def _tc_mid(s, g, b, W, degp):
    def body(s_ref, g_ref, b_ref, w_ref, p_ref, o_ref):
        dinv = _dinv_from_parts(p_ref)
        h = jnp.maximum(dinv * (s_ref[0] + s_ref[1] + g_ref[...]) + b_ref[...],
                        0.0)
        o_ref[...] = jnp.dot(
            h, w_ref[...], preferred_element_type=jnp.float32) * dinv

    return pl.pallas_call(
        body, out_shape=jax.ShapeDtypeStruct((N, D), jnp.float32))(
            s, g, b, W, degp)


def _tc_last(s, g, b, degp):
    def body(s_ref, g_ref, b_ref, p_ref, o_ref):
        dinv = _dinv_from_parts(p_ref)
        o_ref[...] = dinv * (s_ref[0] + s_ref[1] + g_ref[...]) + b_ref[...]

    return pl.pallas_call(
        body, out_shape=jax.ShapeDtypeStruct((N, D), jnp.float32))(s, g, b, degp)


def kernel(x, edge_index, W0, b0, W1, b1, W2, b2):
    src = edge_index[0]
    dst = edge_index[1]
    pad = E_PAD - E
    pad_dst = N + (jnp.arange(pad, dtype=dst.dtype) % 128)
    src_p = jnp.concatenate([src, jnp.zeros((pad,), src.dtype)]
                            ).reshape(NW, STEPS, PB)
    dst_p = jnp.concatenate([dst, pad_dst]).reshape(NW, STEPS, PB)

    degp = _sc_degree(dst_p)
    h0 = _tc_first(x, W0, b0.reshape(1, D))
    g1 = _tc_scale(h0, W1, degp)
    s1 = _sc_segsum(g1, src_p, dst_p)
    g2 = _tc_mid(s1, g1, b1.reshape(1, D), W2, degp)
    s2 = _sc_segsum(g2, src_p, dst_p)
    return _tc_last(s2, g2, b2.reshape(1, D), degp)


# speedup vs baseline: 3.0689x; 3.0482x over previous
import functools

import jax
import jax.numpy as jnp
from jax import lax
from jax.experimental import pallas as pl
from jax.experimental.pallas import tpu as pltpu
from jax.experimental.pallas import tpu_sc as plsc

N = 10000
E = 320000
D = 128

NC = 2
NS = 16
NW = NC * NS

PB = 128
STEPS = 80
E_PAD = NW * STEPS * PB
SCHUNK = 40
N_ACC = 10240
ZROWS = N_ACC // NS
DROWS = 624
DTAIL = N - NS * DROWS

_mesh = plsc.VectorSubcoreMesh(core_axis_name="c", subcore_axis_name="s")


@functools.partial(
    pl.kernel,
    mesh=_mesh,
    out_type=jax.ShapeDtypeStruct((NC, N, 16), jnp.float32),
    scratch_types=[
        pltpu.VMEM((STEPS, PB), jnp.int32),
        pltpu.VMEM((PB, 16), jnp.float32),
        pltpu.VMEM((PB, 16), jnp.float32),
        pltpu.VMEM_SHARED((N_ACC, 16), jnp.float32),
        pltpu.SemaphoreType.DMA,
    ],
)
def _sc_degree(dst_hbm, out_hbm, dst_v, ones_v, zero_v, acc, sem):
    cid = lax.axis_index("c")
    sid = lax.axis_index("s")
    wid = sid * NC + cid

    pltpu.async_copy(dst_hbm.at[wid], dst_v, sem)

    one = jnp.full((16,), 1.0, jnp.float32)
    zero = jnp.zeros((16,), jnp.float32)

    @pl.loop(0, PB)
    def _(i):
        ones_v[i, pl.ds(0, 16)] = one
        zero_v[i, pl.ds(0, 16)] = zero

    zbase = sid * ZROWS

    @pl.loop(0, ZROWS, step=PB)
    def _(r):
        pltpu.sync_copy(zero_v, acc.at[pl.ds(zbase + r, PB)])

    pltpu.make_async_copy(dst_hbm.at[wid], dst_v, sem).wait()
    plsc.subcore_barrier()

    @pl.loop(0, STEPS)
    def _(s):
        pltpu.sync_copy(ones_v, acc.at[dst_v.at[s]], add=True)

    plsc.subcore_barrier()
    dbase = sid * DROWS
    pltpu.sync_copy(acc.at[pl.ds(dbase, DROWS)],
                    out_hbm.at[cid].at[pl.ds(dbase, DROWS)])

    @pl.when(sid == 0)
    def _():
        pltpu.sync_copy(acc.at[pl.ds(NS * DROWS, DTAIL)],
                        out_hbm.at[cid].at[pl.ds(NS * DROWS, DTAIL)])


@functools.partial(
    pl.kernel,
    mesh=_mesh,
    out_type=jax.ShapeDtypeStruct((NC, N, D), jnp.float32),
    scratch_types=[
        pltpu.VMEM((SCHUNK, PB), jnp.int32),
        pltpu.VMEM((SCHUNK, PB), jnp.int32),
        pltpu.VMEM((PB, D), jnp.float32),
        pltpu.VMEM((PB, D), jnp.float32),
        pltpu.VMEM_SHARED((N_ACC, D), jnp.float32),
        pltpu.SemaphoreType.DMA,
        pltpu.SemaphoreType.DMA,
        pltpu.SemaphoreType.DMA,
    ],
)
def _sc_segsum(g_hbm, src_hbm, dst_hbm, out_hbm,
               src_v, dst_v, buf0, buf1, acc, sem0, sem1, semi):
    cid = lax.axis_index("c")
    sid = lax.axis_index("s")
    wid = sid * NC + cid

    zero = jnp.zeros((16,), jnp.float32)

    @pl.loop(0, PB)
    def _(i):
        @pl.loop(0, D, step=16)
        def _(j):
            buf0[i, pl.ds(j, 16)] = zero

    zbase = sid * ZROWS

    @pl.loop(0, ZROWS, step=PB)
    def _(r):
        pltpu.sync_copy(buf0, acc.at[pl.ds(zbase + r, PB)])

    plsc.subcore_barrier()

    def process_chunk(src_chunk_hbm, dst_chunk_hbm):
        pltpu.async_copy(src_chunk_hbm, src_v, semi)
        pltpu.async_copy(dst_chunk_hbm, dst_v, semi)
        pltpu.make_async_copy(src_chunk_hbm, src_v, semi).wait()
        pltpu.make_async_copy(dst_chunk_hbm, dst_v, semi).wait()

        pltpu.async_copy(g_hbm.at[src_v.at[0]], buf0, sem0)

        @pl.loop(0, SCHUNK - 2, step=2)
        def _(s):
            pltpu.async_copy(g_hbm.at[src_v.at[s + 1]], buf1, sem1)
            pltpu.make_async_copy(g_hbm.at[src_v.at[s]], buf0, sem0).wait()
            pltpu.sync_copy(buf0, acc.at[dst_v.at[s]], add=True)
            pltpu.async_copy(g_hbm.at[src_v.at[s + 2]], buf0, sem0)
            pltpu.make_async_copy(g_hbm.at[src_v.at[s + 1]], buf1, sem1).wait()
            pltpu.sync_copy(buf1, acc.at[dst_v.at[s + 1]], add=True)

        pltpu.async_copy(g_hbm.at[src_v.at[SCHUNK - 1]], buf1, sem1)
        pltpu.make_async_copy(g_hbm.at[src_v.at[SCHUNK - 2]], buf0, sem0).wait()
        pltpu.sync_copy(buf0, acc.at[dst_v.at[SCHUNK - 2]], add=True)
        pltpu.make_async_copy(g_hbm.at[src_v.at[SCHUNK - 1]], buf1, sem1).wait()
        pltpu.sync_copy(buf1, acc.at[dst_v.at[SCHUNK - 1]], add=True)

    for h in range(STEPS // SCHUNK):
        process_chunk(src_hbm.at[wid].at[pl.ds(h * SCHUNK, SCHUNK)],
                      dst_hbm.at[wid].at[pl.ds(h * SCHUNK, SCHUNK)])

    plsc.subcore_barrier()
    dbase = sid * DROWS
    pltpu.sync_copy(acc.at[pl.ds(dbase, DROWS)],
                    out_hbm.at[cid].at[pl.ds(dbase, DROWS)])

    @pl.when(sid == 0)
    def _():
        pltpu.sync_copy(acc.at[pl.ds(NS * DROWS, DTAIL)],
                        out_hbm.at[cid].at[pl.ds(NS * DROWS, DTAIL)])


def _tc_first(x, W0, b0):
    def body(x_ref, w_ref, b_ref, o_ref):
        o_ref[...] = jnp.maximum(
            jnp.dot(x_ref[...], w_ref[...], preferred_element_type=jnp.float32)
            + b_ref[...], 0.0)

    return pl.pallas_call(
        body, out_shape=jax.ShapeDtypeStruct((N, D), jnp.float32))(x, W0, b0)


def _dinv_from_parts(p_ref):
    deg = 1.0 + p_ref[0][:, 0:1] + p_ref[1][:, 0:1]
    return 1.0 / jnp.sqrt(deg)


def _tc_scale(h, W, degp):
    def body(h_ref, w_ref, p_ref, o_ref):
        dinv = _dinv_from_parts(p_ref)
        o_ref[...] = jnp.dot(
            h_ref[...], w_ref[...], preferred_element_type=jnp.float32) * dinv

    return pl.pallas_call(
        body, out_shape=jax.ShapeDtypeStruct((N, D), jnp.float32))(h, W, degp)


def _tc_mid(s, g, b, W, degp):
    def body(s_ref, g_ref, b_ref, w_ref, p_ref, o_ref):
        dinv = _dinv_from_parts(p_ref)
        h = jnp.maximum(dinv * (s_ref[0] + s_ref[1] + g_ref[...]) + b_ref[...],
                        0.0)
        o_ref[...] = jnp.dot(
            h, w_ref[...], preferred_element_type=jnp.float32) * dinv

    return pl.pallas_call(
        body, out_shape=jax.ShapeDtypeStruct((N, D), jnp.float32))(
            s, g, b, W, degp)


def _tc_last(s, g, b, degp):
    def body(s_ref, g_ref, b_ref, p_ref, o_ref):
        dinv = _dinv_from_parts(p_ref)
        o_ref[...] = dinv * (s_ref[0] + s_ref[1] + g_ref[...]) + b_ref[...]

    return pl.pallas_call(
        body, out_shape=jax.ShapeDtypeStruct((N, D), jnp.float32))(s, g, b, degp)


def kernel(x, edge_index, W0, b0, W1, b1, W2, b2):
    src = edge_index[0]
    dst = edge_index[1]
    pad = E_PAD - E
    pad_iota = jnp.arange(pad, dtype=dst.dtype) % 128
    pad_dst = N + pad_iota
    src_p = jnp.concatenate([src, pad_iota]).reshape(NW, STEPS, PB)
    dst_p = jnp.concatenate([dst, pad_dst]).reshape(NW, STEPS, PB)

    degp = _sc_degree(dst_p)
    h0 = _tc_first(x, W0, b0.reshape(1, D))
    g1 = _tc_scale(h0, W1, degp)
    s1 = _sc_segsum(g1, src_p, dst_p)
    g2 = _tc_mid(s1, g1, b1.reshape(1, D), W2, degp)
    s2 = _sc_segsum(g2, src_p, dst_p)
    return _tc_last(s2, g2, b2.reshape(1, D), degp)
